# Initial kernel scaffold; baseline (speedup 1.0000x reference)
import jax
import jax.numpy as jnp
from jax.experimental import pallas as pl


def kernel(t, alphas_cumprod):
    raise NotImplementedError("write your pallas kernel here")


# speedup vs baseline: 4.5199x; 4.5199x over previous
import functools

import jax
import jax.numpy as jnp
from jax import lax
from jax.experimental import pallas as pl
from jax.experimental.pallas import tpu as pltpu
from jax.experimental.pallas import tpu_sc as plsc

_NUM_TIMESTEPS = 1000
_TABLE_LEN = _NUM_TIMESTEPS + 1
_BATCH = 16384
_NC = 2
_NS = 16
_L = 16
_NW = _NC * _NS
_B_PER_W = _BATCH // _NW

_mesh = plsc.VectorSubcoreMesh(core_axis_name="c", subcore_axis_name="s")


@functools.partial(
    pl.kernel,
    mesh=_mesh,
    out_type=jax.ShapeDtypeStruct((_BATCH,), jnp.float32),
    scratch_types=[
        pltpu.VMEM((_TABLE_LEN,), jnp.float32),
        pltpu.VMEM((_B_PER_W,), jnp.int32),
        pltpu.VMEM((_B_PER_W,), jnp.float32),
    ],
    compiler_params=pltpu.CompilerParams(needs_layout_passes=False),
)
def _alpha_bar_gather(t_hbm, table_hbm, out_hbm, table_v, idx_v, res_v):
    wid = lax.axis_index("s") * _NC + lax.axis_index("c")
    base = wid * _B_PER_W
    pltpu.sync_copy(table_hbm, table_v)
    pltpu.sync_copy(t_hbm.at[pl.ds(base, _B_PER_W)], idx_v)
    for i in range(_B_PER_W // _L):
        idx = idx_v[pl.ds(i * _L, _L)]
        idx = jnp.minimum(jnp.maximum(idx, 0), _NUM_TIMESTEPS - 1)
        res_v[pl.ds(i * _L, _L)] = plsc.load_gather(table_v, [idx])
    pltpu.sync_copy(res_v, out_hbm.at[pl.ds(base, _B_PER_W)])


def kernel(t, alphas_cumprod):
    return _alpha_bar_gather(t.astype(jnp.int32), alphas_cumprod)


# speedup vs baseline: 4.6919x; 1.0381x over previous
import functools

import jax
import jax.numpy as jnp
from jax import lax
from jax.experimental import pallas as pl
from jax.experimental.pallas import tpu as pltpu
from jax.experimental.pallas import tpu_sc as plsc

_NUM_TIMESTEPS = 1000
_TABLE_LEN = _NUM_TIMESTEPS + 1
_BATCH = 16384
_NC = 2
_NS = 16
_L = 16
_NW = _NC * _NS
_B_PER_W = _BATCH // _NW

_mesh = plsc.VectorSubcoreMesh(core_axis_name="c", subcore_axis_name="s")


@functools.partial(
    pl.kernel,
    mesh=_mesh,
    out_type=jax.ShapeDtypeStruct((_BATCH,), jnp.float32),
    scratch_types=[
        pltpu.VMEM((_TABLE_LEN,), jnp.float32),
        pltpu.VMEM((_B_PER_W,), jnp.int32),
        pltpu.VMEM((_B_PER_W,), jnp.float32),
        pltpu.SemaphoreType.DMA,
        pltpu.SemaphoreType.DMA,
        pltpu.SemaphoreType.DMA,
    ],
    compiler_params=pltpu.CompilerParams(needs_layout_passes=False),
)
def _alpha_bar_gather(t_hbm, table_hbm, out_hbm, table_v, idx_v, res_v,
                      tsem, isem, osem):
    wid = lax.axis_index("s") * _NC + lax.axis_index("c")
    base = wid * _B_PER_W
    half = _B_PER_W // 2
    tcopy = pltpu.async_copy(table_hbm, table_v, tsem)
    icopy = pltpu.async_copy(t_hbm.at[pl.ds(base, _B_PER_W)], idx_v, isem)
    tcopy.wait()
    icopy.wait()
    for i in range(half // _L):
        idx = idx_v[pl.ds(i * _L, _L)]
        idx = jnp.minimum(jnp.maximum(idx, 0), _NUM_TIMESTEPS - 1)
        res_v[pl.ds(i * _L, _L)] = plsc.load_gather(table_v, [idx])
    ocopy0 = pltpu.async_copy(
        res_v.at[pl.ds(0, half)], out_hbm.at[pl.ds(base, half)], osem)
    for i in range(half // _L, _B_PER_W // _L):
        idx = idx_v[pl.ds(i * _L, _L)]
        idx = jnp.minimum(jnp.maximum(idx, 0), _NUM_TIMESTEPS - 1)
        res_v[pl.ds(i * _L, _L)] = plsc.load_gather(table_v, [idx])
    ocopy1 = pltpu.async_copy(
        res_v.at[pl.ds(half, half)], out_hbm.at[pl.ds(base + half, half)], osem)
    ocopy0.wait()
    ocopy1.wait()


def kernel(t, alphas_cumprod):
    return _alpha_bar_gather(t.astype(jnp.int32), alphas_cumprod)


# speedup vs baseline: 5.0393x; 1.0740x over previous
import functools

import jax
import jax.numpy as jnp
from jax import lax
from jax.experimental import pallas as pl
from jax.experimental.pallas import tpu as pltpu
from jax.experimental.pallas import tpu_sc as plsc

_NUM_TIMESTEPS = 1000
_TABLE_LEN = _NUM_TIMESTEPS + 1
_BATCH = 16384
_NC = 1
_NS = 16
_L = 16
_NW = _NC * _NS
_B_PER_W = _BATCH // _NW

_mesh = plsc.VectorSubcoreMesh(
    core_axis_name="c", subcore_axis_name="s", num_cores=_NC)


@functools.partial(
    pl.kernel,
    mesh=_mesh,
    out_type=jax.ShapeDtypeStruct((_BATCH,), jnp.float32),
    scratch_types=[
        pltpu.VMEM((_TABLE_LEN,), jnp.float32),
        pltpu.VMEM((_B_PER_W,), jnp.int32),
        pltpu.VMEM((_B_PER_W,), jnp.float32),
        pltpu.SemaphoreType.DMA,
        pltpu.SemaphoreType.DMA,
        pltpu.SemaphoreType.DMA,
    ],
    compiler_params=pltpu.CompilerParams(needs_layout_passes=False),
)
def _alpha_bar_gather(t_hbm, table_hbm, out_hbm, table_v, idx_v, res_v,
                      tsem, isem, osem):
    wid = lax.axis_index("s") * _NC + lax.axis_index("c")
    base = wid * _B_PER_W
    half = _B_PER_W // 2
    tcopy = pltpu.async_copy(table_hbm, table_v, tsem)
    icopy = pltpu.async_copy(t_hbm.at[pl.ds(base, _B_PER_W)], idx_v, isem)
    tcopy.wait()
    icopy.wait()
    for i in range(half // _L):
        idx = idx_v[pl.ds(i * _L, _L)]
        idx = jnp.minimum(jnp.maximum(idx, 0), _NUM_TIMESTEPS - 1)
        res_v[pl.ds(i * _L, _L)] = plsc.load_gather(table_v, [idx])
    ocopy0 = pltpu.async_copy(
        res_v.at[pl.ds(0, half)], out_hbm.at[pl.ds(base, half)], osem)
    for i in range(half // _L, _B_PER_W // _L):
        idx = idx_v[pl.ds(i * _L, _L)]
        idx = jnp.minimum(jnp.maximum(idx, 0), _NUM_TIMESTEPS - 1)
        res_v[pl.ds(i * _L, _L)] = plsc.load_gather(table_v, [idx])
    ocopy1 = pltpu.async_copy(
        res_v.at[pl.ds(half, half)], out_hbm.at[pl.ds(base + half, half)], osem)
    ocopy0.wait()
    ocopy1.wait()


def kernel(t, alphas_cumprod):
    return _alpha_bar_gather(t.astype(jnp.int32), alphas_cumprod)
